# baseline (speedup 1.0000x reference)
import functools

import jax
import jax.numpy as jnp
from jax import lax
from jax.experimental import pallas as pl
from jax.experimental.pallas import tpu as pltpu
from jax.experimental.pallas import tpu_sc as plsc

NC = 2
NS = 16
NW = NC * NS
LANES = 16


def _deg_kernel_body(n, e, chunk, dst_hbm, w_hbm, out_hbm, deg_v, idx_v, wv_v):
    c = lax.axis_index("c")
    s = lax.axis_index("s")
    wid = s * NC + c
    epw = e // NW

    zero16 = jnp.zeros((LANES,), jnp.float32)
    unz = 5
    unr = 5

    def zbody(i, _):
        for u in range(unz):
            deg_v[pl.ds((i * unz + u) * LANES, LANES)] = zero16
        return ()

    lax.fori_loop(0, n // (LANES * unz), zbody, ())

    base0 = wid * epw

    def obody(i, _):
        base = base0 + i * chunk
        pltpu.sync_copy(dst_hbm.at[pl.ds(base, chunk)], idx_v)
        pltpu.sync_copy(w_hbm.at[pl.ds(base, chunk)], wv_v)

        def ibody(j, _):
            for u in range(unr):
                off = (j * unr + u) * LANES
                idx = idx_v[pl.ds(off, LANES)]
                wv = wv_v[pl.ds(off, LANES)]
                plsc.addupdate_scatter(deg_v, [idx], wv)
            return ()

        lax.fori_loop(0, chunk // (LANES * unr), ibody, ())
        return ()

    lax.fori_loop(0, epw // chunk, obody, ())
    pltpu.sync_copy(deg_v, out_hbm.at[pl.ds(wid * n, n)])


def _deg_partials(dst, w, n):
    e = w.shape[0]
    chunk = 400
    assert e % NW == 0 and (e // NW) % chunk == 0 and n % LANES == 0
    mesh = plsc.VectorSubcoreMesh(core_axis_name="c", subcore_axis_name="s")
    k = pl.kernel(
        functools.partial(_deg_kernel_body, n, e, chunk),
        out_type=jax.ShapeDtypeStruct((NW * n,), jnp.float32),
        mesh=mesh,
        compiler_params=pltpu.CompilerParams(needs_layout_passes=False),
        scratch_types=[
            pltpu.VMEM((n,), jnp.float32),
            pltpu.VMEM((chunk,), jnp.int32),
            pltpu.VMEM((chunk,), jnp.float32),
        ],
    )
    return k(dst, w).reshape(NW, n)


def _matmul_g_body(x_ref, w_ref, deg_ref, g_ref):
    h = jnp.dot(x_ref[...], w_ref[...], preferred_element_type=jnp.float32)
    deg = jnp.sum(deg_ref[...], axis=0) + 1.0
    dinv = lax.rsqrt(deg)
    g_ref[...] = h * dinv[:, None]


def _matmul_g(emb, w, deg_parts):
    n, d = emb.shape
    return pl.pallas_call(
        _matmul_g_body,
        out_shape=jax.ShapeDtypeStruct((n, d), jnp.float32),
    )(emb, w, deg_parts)


def _edge_kernel_body(n_pad, d, e, kb, src_hbm, dst_hbm, dst2_hbm, w_hbm, g_hbm,
                      out_hbm, src_sl, dst_sl, w_sl, rows0_v, rows1_v, dstw_v,
                      dstc_v, zbuf_v, acc_sh, sem0, sem1, ssem0, ssem1):
    c = lax.axis_index("c")
    s = lax.axis_index("s")
    rows_per_tile = n_pad // NS
    zrows = zbuf_v.shape[0]
    iota = lax.iota(jnp.int32, LANES)

    zero16 = jnp.zeros((LANES,), jnp.float32)

    def zbody(i, _):
        zbuf_v[i // (d // LANES), pl.ds((i % (d // LANES)) * LANES, LANES)] = zero16
        return ()

    lax.fori_loop(0, zrows * d // LANES, zbody, ())

    def zcopy(i, _):
        pltpu.sync_copy(zbuf_v, acc_sh.at[pl.ds(s * rows_per_tile + i * zrows, zrows)])
        return ()

    lax.fori_loop(0, rows_per_tile // zrows, zcopy, ())
    plsc.subcore_barrier()

    epc = e // NC
    epw = epc // NS
    base0 = c * epc + s * epw
    csz = src_sl.shape[0]

    nb = csz // kb

    nb = csz // kb
    zero16i = jnp.zeros((LANES,), jnp.int32)

    def process(rows_v, ssem, loff):
        for q in range(kb // LANES):
            goff = loff + q * LANES
            for j in range(LANES):
                wj = plsc.load_gather(
                    w_sl, [jnp.full((LANES,), goff + j, jnp.int32)])
                r = q * LANES + j
                for ch in range(d // LANES):
                    sl = pl.ds(ch * LANES, LANES)
                    rows_v[r, sl] = rows_v[r, sl] * wj
            dst16 = dst_sl[pl.ds(goff, LANES)]
            cnt, _ = plsc.scan_count(dst16)
            has_dup = jnp.max(cnt) != jnp.min(cnt)

            def fast():
                dstw_v[q, :] = dst16
                pltpu.async_copy(rows_v.at[pl.ds(q * LANES, LANES)],
                                 acc_sh.at[dstw_v.at[q]], ssem, add=True)

            def slow():
                plsc.store_scatter(dstc_v.at[q], [iota, zero16i], dst16)
                for j in range(LANES):
                    pltpu.async_copy(rows_v.at[pl.ds(q * LANES + j, 1)],
                                     acc_sh.at[dstc_v.at[q, j]], ssem, add=True)

            lax.cond(has_dup, slow, fast)

    def prefetch(i, rows_v, sem):
        loff = jnp.minimum(i, nb - 1) * kb
        pltpu.async_copy(g_hbm.at[src_sl.at[pl.ds(loff, kb)]], rows_v, sem)

    def gwait(rows_v, sem):
        pltpu.make_async_copy(g_hbm.at[pl.ds(0, kb)], rows_v, sem).wait()

    def sdrain(rows_v, ssem):
        pltpu.make_async_copy(g_hbm.at[pl.ds(0, kb)], rows_v, ssem).wait()

    def cbody(ci, _):
        cb = base0 + ci * csz
        pltpu.sync_copy(src_hbm.at[pl.ds(cb, csz)], src_sl)
        pltpu.sync_copy(dst_hbm.at[pl.ds(cb, csz)], dst_sl)
        pltpu.sync_copy(w_hbm.at[pl.ds(cb, csz)], w_sl)
        prefetch(jnp.int32(0), rows0_v, sem0)

        def ebatch(i, _):
            @pl.when(i % 2 == 0)
            def _():
                @pl.when(i >= 1)
                def _():
                    sdrain(rows1_v, ssem1)

                prefetch(i + 1, rows1_v, sem1)
                gwait(rows0_v, sem0)
                process(rows0_v, ssem0, i * kb)

            @pl.when(i % 2 == 1)
            def _():
                sdrain(rows0_v, ssem0)
                prefetch(i + 1, rows0_v, sem0)
                gwait(rows1_v, sem1)
                process(rows1_v, ssem1, i * kb)

            return ()

        lax.fori_loop(0, nb, ebatch, ())
        if nb % 2 == 0:
            sdrain(rows1_v, ssem1)
            gwait(rows0_v, sem0)
        else:
            sdrain(rows0_v, ssem0)
            gwait(rows1_v, sem1)
        return ()

    lax.fori_loop(0, epw // csz, cbody, ())
    plsc.subcore_barrier()
    pltpu.sync_copy(
        acc_sh.at[pl.ds(s * rows_per_tile, rows_per_tile)],
        out_hbm.at[c, pl.ds(s * rows_per_tile, rows_per_tile)],
    )


def _edge_partials(src, dst, w, g):
    n, d = g.shape
    e = w.shape[0]
    kb = 80
    zrows = 32
    csz = 2000
    n_pad = ((n + 128 * NS - 1) // (128 * NS)) * (128 * NS)
    assert e % NC == 0 and (e // NC) % NS == 0 and (e // NW) % csz == 0
    assert csz % kb == 0 and (n_pad // NS) % zrows == 0
    mesh = plsc.VectorSubcoreMesh(core_axis_name="c", subcore_axis_name="s")
    k = pl.kernel(
        functools.partial(_edge_kernel_body, n_pad, d, e, kb),
        out_type=jax.ShapeDtypeStruct((NC, n_pad, d), jnp.float32),
        mesh=mesh,
        compiler_params=pltpu.CompilerParams(needs_layout_passes=False),
        scratch_types=[
            pltpu.VMEM((csz,), jnp.int32),
            pltpu.VMEM((csz,), jnp.int32),
            pltpu.VMEM((csz,), jnp.float32),
            pltpu.VMEM((kb, d), jnp.float32),
            pltpu.VMEM((kb, d), jnp.float32),
            pltpu.VMEM((kb // LANES, LANES), jnp.int32),
            pltpu.VMEM((kb // LANES, LANES, 1), jnp.int32),
            pltpu.VMEM((zrows, d), jnp.float32),
            pltpu.VMEM_SHARED((n_pad, d), jnp.float32),
            pltpu.SemaphoreType.DMA,
            pltpu.SemaphoreType.DMA,
            pltpu.SemaphoreType.DMA,
            pltpu.SemaphoreType.DMA,
        ],
    )
    return k(src, dst, dst.reshape(e, 1), w, g)


def _combine_body(n, p_ref, g_ref, deg_ref, b_ref, o_ref):
    deg = jnp.sum(deg_ref[...], axis=0) + 1.0
    dinv = lax.rsqrt(deg)
    ssum = p_ref[0, pl.ds(0, n), :] + p_ref[1, pl.ds(0, n), :] + g_ref[...]
    o_ref[...] = ssum * dinv[:, None] + b_ref[...]


def _combine(p, g, deg_parts, b):
    n, d = g.shape
    return pl.pallas_call(
        functools.partial(_combine_body, n),
        out_shape=jax.ShapeDtypeStruct((n, d), jnp.float32),
    )(p, g, deg_parts, b)


def kernel(x, edge_index, weights, emb_table, W, b):
    n, d = emb_table.shape
    src = edge_index[0]
    dst = edge_index[1]
    deg_parts = _deg_partials(dst, weights, n)
    g = _matmul_g(emb_table, W, deg_parts)
    p = _edge_partials(src, dst, weights, g)
    out = _combine(p, g, deg_parts, b.reshape(1, d))
    return out[None, :, None, :]


# speedup vs baseline: 1.7070x; 1.7070x over previous
import functools

import jax
import jax.numpy as jnp
from jax import lax
from jax.experimental import pallas as pl
from jax.experimental.pallas import tpu as pltpu
from jax.experimental.pallas import tpu_sc as plsc

NC = 2
NS = 16
NW = NC * NS
LANES = 16


def _deg_kernel_body(n, e, chunk, dst_hbm, w_hbm, out_hbm, deg_v, idx_v, wv_v):
    c = lax.axis_index("c")
    s = lax.axis_index("s")
    wid = s * NC + c
    epw = e // NW

    zero16 = jnp.zeros((LANES,), jnp.float32)
    unz = 5
    unr = 5

    def zbody(i, _):
        for u in range(unz):
            deg_v[pl.ds((i * unz + u) * LANES, LANES)] = zero16
        return ()

    lax.fori_loop(0, n // (LANES * unz), zbody, ())

    base0 = wid * epw

    def obody(i, _):
        base = base0 + i * chunk
        pltpu.sync_copy(dst_hbm.at[pl.ds(base, chunk)], idx_v)
        pltpu.sync_copy(w_hbm.at[pl.ds(base, chunk)], wv_v)

        def ibody(j, _):
            for u in range(unr):
                off = (j * unr + u) * LANES
                idx = idx_v[pl.ds(off, LANES)]
                wv = wv_v[pl.ds(off, LANES)]
                plsc.addupdate_scatter(deg_v, [idx], wv)
            return ()

        lax.fori_loop(0, chunk // (LANES * unr), ibody, ())
        return ()

    lax.fori_loop(0, epw // chunk, obody, ())
    pltpu.sync_copy(deg_v, out_hbm.at[pl.ds(wid * n, n)])


def _deg_partials(dst, w, n):
    e = w.shape[0]
    chunk = 2000
    assert e % NW == 0 and (e // NW) % chunk == 0 and n % LANES == 0
    mesh = plsc.VectorSubcoreMesh(core_axis_name="c", subcore_axis_name="s")
    k = pl.kernel(
        functools.partial(_deg_kernel_body, n, e, chunk),
        out_type=jax.ShapeDtypeStruct((NW * n,), jnp.float32),
        mesh=mesh,
        compiler_params=pltpu.CompilerParams(needs_layout_passes=False),
        scratch_types=[
            pltpu.VMEM((n,), jnp.float32),
            pltpu.VMEM((chunk,), jnp.int32),
            pltpu.VMEM((chunk,), jnp.float32),
        ],
    )
    return k(dst, w).reshape(NW, n)


def _matmul_g_body(x_ref, w_ref, deg_ref, g_ref):
    h = jnp.dot(x_ref[...], w_ref[...], preferred_element_type=jnp.float32)
    deg = jnp.sum(deg_ref[...], axis=0) + 1.0
    dinv = lax.rsqrt(deg)
    g_ref[...] = h * dinv[:, None]


def _matmul_g(emb, w, deg_parts):
    n, d = emb.shape
    return pl.pallas_call(
        _matmul_g_body,
        out_shape=jax.ShapeDtypeStruct((n, d), jnp.float32),
    )(emb, w, deg_parts)


def _edge_kernel_body(n_pad, d, e, kb, src_hbm, dst_hbm, w_hbm, g_hbm,
                      out_hbm, src_sl, dst_sl, w_sl, rows0_v, rows1_v, dstw_v,
                      dstc_v, zbuf_v, acc_sh, sem0, sem1, ssem0, ssem1):
    c = lax.axis_index("c")
    s = lax.axis_index("s")
    rows_per_tile = n_pad // NS
    zrows = zbuf_v.shape[0]
    iota = lax.iota(jnp.int32, LANES)

    zero16 = jnp.zeros((LANES,), jnp.float32)

    def zbody(i, _):
        zbuf_v[i // (d // LANES), pl.ds((i % (d // LANES)) * LANES, LANES)] = zero16
        return ()

    lax.fori_loop(0, zrows * d // LANES, zbody, ())

    def zcopy(i, _):
        pltpu.sync_copy(zbuf_v, acc_sh.at[pl.ds(s * rows_per_tile + i * zrows, zrows)])
        return ()

    lax.fori_loop(0, rows_per_tile // zrows, zcopy, ())
    plsc.subcore_barrier()

    epc = e // NC
    epw = epc // NS
    base0 = c * epc + s * epw
    csz = src_sl.shape[0]

    nb = csz // kb

    nb = csz // kb
    zero16i = jnp.zeros((LANES,), jnp.int32)

    def process(rows_v, ssem, loff):
        for q in range(kb // LANES):
            goff = loff + q * LANES
            w16 = w_sl[pl.ds(goff, LANES)]
            for j in range(LANES):
                wj = jnp.full((LANES,), w16[j])
                r = q * LANES + j
                for ch in range(d // LANES):
                    sl = pl.ds(ch * LANES, LANES)
                    rows_v[r, sl] = rows_v[r, sl] * wj
            dst16 = dst_sl[pl.ds(goff, LANES)]
            cnt, _ = plsc.scan_count(dst16)
            has_dup = jnp.max(cnt) != jnp.min(cnt)

            def fast():
                dstw_v[q, :] = dst16
                pltpu.async_copy(rows_v.at[pl.ds(q * LANES, LANES)],
                                 acc_sh.at[dstw_v.at[q]], ssem, add=True)

            def slow():
                plsc.store_scatter(dstc_v.at[q], [iota, zero16i], dst16)
                for j in range(LANES):
                    pltpu.async_copy(rows_v.at[pl.ds(q * LANES + j, 1)],
                                     acc_sh.at[dstc_v.at[q, j]], ssem, add=True)

            lax.cond(has_dup, slow, fast)

    def prefetch(i, rows_v, sem):
        loff = jnp.minimum(i, nb - 1) * kb
        pltpu.async_copy(g_hbm.at[src_sl.at[pl.ds(loff, kb)]], rows_v, sem)

    def gwait(rows_v, sem):
        pltpu.make_async_copy(g_hbm.at[pl.ds(0, kb)], rows_v, sem).wait()

    def sdrain(rows_v, ssem):
        pltpu.make_async_copy(g_hbm.at[pl.ds(0, kb)], rows_v, ssem).wait()

    def cbody(ci, _):
        cb = base0 + ci * csz
        pltpu.sync_copy(src_hbm.at[pl.ds(cb, csz)], src_sl)
        pltpu.sync_copy(dst_hbm.at[pl.ds(cb, csz)], dst_sl)
        pltpu.sync_copy(w_hbm.at[pl.ds(cb, csz)], w_sl)
        prefetch(jnp.int32(0), rows0_v, sem0)

        def ebatch(i, _):
            @pl.when(i % 2 == 0)
            def _():
                @pl.when(i >= 1)
                def _():
                    sdrain(rows1_v, ssem1)

                prefetch(i + 1, rows1_v, sem1)
                gwait(rows0_v, sem0)
                process(rows0_v, ssem0, i * kb)

            @pl.when(i % 2 == 1)
            def _():
                sdrain(rows0_v, ssem0)
                prefetch(i + 1, rows0_v, sem0)
                gwait(rows1_v, sem1)
                process(rows1_v, ssem1, i * kb)

            return ()

        lax.fori_loop(0, nb, ebatch, ())
        if nb % 2 == 0:
            sdrain(rows1_v, ssem1)
            gwait(rows0_v, sem0)
        else:
            sdrain(rows0_v, ssem0)
            gwait(rows1_v, sem1)
        return ()

    lax.fori_loop(0, epw // csz, cbody, ())
    plsc.subcore_barrier()
    pltpu.sync_copy(
        acc_sh.at[pl.ds(s * rows_per_tile, rows_per_tile)],
        out_hbm.at[c, pl.ds(s * rows_per_tile, rows_per_tile)],
    )


def _edge_partials(src, dst, w, g):
    n, d = g.shape
    e = w.shape[0]
    kb = 80
    zrows = 32
    csz = 2000
    n_pad = ((n + 128 * NS - 1) // (128 * NS)) * (128 * NS)
    assert e % NC == 0 and (e // NC) % NS == 0 and (e // NW) % csz == 0
    assert csz % kb == 0 and (n_pad // NS) % zrows == 0
    mesh = plsc.VectorSubcoreMesh(core_axis_name="c", subcore_axis_name="s")
    k = pl.kernel(
        functools.partial(_edge_kernel_body, n_pad, d, e, kb),
        out_type=jax.ShapeDtypeStruct((NC, n_pad, d), jnp.float32),
        mesh=mesh,
        compiler_params=pltpu.CompilerParams(needs_layout_passes=False),
        scratch_types=[
            pltpu.VMEM((csz,), jnp.int32),
            pltpu.VMEM((csz,), jnp.int32),
            pltpu.VMEM((csz,), jnp.float32),
            pltpu.VMEM((kb, d), jnp.float32),
            pltpu.VMEM((kb, d), jnp.float32),
            pltpu.VMEM((kb // LANES, LANES), jnp.int32),
            pltpu.VMEM((kb // LANES, LANES, 1), jnp.int32),
            pltpu.VMEM((zrows, d), jnp.float32),
            pltpu.VMEM_SHARED((n_pad, d), jnp.float32),
            pltpu.SemaphoreType.DMA,
            pltpu.SemaphoreType.DMA,
            pltpu.SemaphoreType.DMA,
            pltpu.SemaphoreType.DMA,
        ],
    )
    return k(src, dst, w, g)


def _combine_body(n, p_ref, g_ref, deg_ref, b_ref, o_ref):
    deg = jnp.sum(deg_ref[...], axis=0) + 1.0
    dinv = lax.rsqrt(deg)
    ssum = p_ref[0, pl.ds(0, n), :] + p_ref[1, pl.ds(0, n), :] + g_ref[...]
    o_ref[...] = ssum * dinv[:, None] + b_ref[...]


def _combine(p, g, deg_parts, b):
    n, d = g.shape
    return pl.pallas_call(
        functools.partial(_combine_body, n),
        out_shape=jax.ShapeDtypeStruct((n, d), jnp.float32),
    )(p, g, deg_parts, b)


def kernel(x, edge_index, weights, emb_table, W, b):
    n, d = emb_table.shape
    src = edge_index[0]
    dst = edge_index[1]
    deg_parts = _deg_partials(dst, weights, n)
    g = _matmul_g(emb_table, W, deg_parts)
    p = _edge_partials(src, dst, weights, g)
    out = _combine(p, g, deg_parts, b.reshape(1, d))
    return out[None, :, None, :]


# speedup vs baseline: 1.7102x; 1.0019x over previous
import functools

import jax
import jax.numpy as jnp
from jax import lax
from jax.experimental import pallas as pl
from jax.experimental.pallas import tpu as pltpu
from jax.experimental.pallas import tpu_sc as plsc

NC = 2
NS = 16
NW = NC * NS
LANES = 16


def _deg_kernel_body(n, e, chunk, dst_hbm, w_hbm, out_hbm, deg_v, idx_v, wv_v):
    c = lax.axis_index("c")
    s = lax.axis_index("s")
    wid = s * NC + c
    epw = e // NW

    zero16 = jnp.zeros((LANES,), jnp.float32)
    unz = 5
    unr = 5

    def zbody(i, _):
        for u in range(unz):
            deg_v[pl.ds((i * unz + u) * LANES, LANES)] = zero16
        return ()

    lax.fori_loop(0, n // (LANES * unz), zbody, ())

    base0 = wid * epw

    def obody(i, _):
        base = base0 + i * chunk
        pltpu.sync_copy(dst_hbm.at[pl.ds(base, chunk)], idx_v)
        pltpu.sync_copy(w_hbm.at[pl.ds(base, chunk)], wv_v)

        def ibody(j, _):
            for u in range(unr):
                off = (j * unr + u) * LANES
                idx = idx_v[pl.ds(off, LANES)]
                wv = wv_v[pl.ds(off, LANES)]
                plsc.addupdate_scatter(deg_v, [idx], wv)
            return ()

        lax.fori_loop(0, chunk // (LANES * unr), ibody, ())
        return ()

    lax.fori_loop(0, epw // chunk, obody, ())
    pltpu.sync_copy(deg_v, out_hbm.at[pl.ds(wid * n, n)])


def _deg_partials(dst, w, n):
    e = w.shape[0]
    chunk = 2000
    assert e % NW == 0 and (e // NW) % chunk == 0 and n % LANES == 0
    mesh = plsc.VectorSubcoreMesh(core_axis_name="c", subcore_axis_name="s")
    k = pl.kernel(
        functools.partial(_deg_kernel_body, n, e, chunk),
        out_type=jax.ShapeDtypeStruct((NW * n,), jnp.float32),
        mesh=mesh,
        compiler_params=pltpu.CompilerParams(needs_layout_passes=False),
        scratch_types=[
            pltpu.VMEM((n,), jnp.float32),
            pltpu.VMEM((chunk,), jnp.int32),
            pltpu.VMEM((chunk,), jnp.float32),
        ],
    )
    return k(dst, w).reshape(NW, n)


def _matmul_g_body(x_ref, w_ref, deg_ref, g_ref):
    h = jnp.dot(x_ref[...], w_ref[...], preferred_element_type=jnp.float32)
    deg = jnp.sum(deg_ref[...], axis=0) + 1.0
    dinv = lax.rsqrt(deg)
    g_ref[...] = h * dinv[:, None]


def _matmul_g(emb, w, deg_parts):
    n, d = emb.shape
    return pl.pallas_call(
        _matmul_g_body,
        out_shape=jax.ShapeDtypeStruct((n, d), jnp.float32),
    )(emb, w, deg_parts)


def _edge_kernel_body(n_pad, d, e, kb, src_hbm, dst_hbm, w_hbm, g_hbm,
                      out_hbm, src_sl, dst_sl, w_sl, rows0_v, rows1_v, dstw_v,
                      dstc_v, zbuf_v, acc_sh, sem0, sem1, ssem0, ssem1):
    c = lax.axis_index("c")
    s = lax.axis_index("s")
    rows_per_tile = n_pad // NS
    zrows = zbuf_v.shape[0]
    iota = lax.iota(jnp.int32, LANES)

    zero16 = jnp.zeros((LANES,), jnp.float32)

    def zbody(i, _):
        zbuf_v[i // (d // LANES), pl.ds((i % (d // LANES)) * LANES, LANES)] = zero16
        return ()

    lax.fori_loop(0, zrows * d // LANES, zbody, ())

    def zcopy(i, _):
        pltpu.sync_copy(zbuf_v, acc_sh.at[pl.ds(s * rows_per_tile + i * zrows, zrows)])
        return ()

    lax.fori_loop(0, rows_per_tile // zrows, zcopy, ())
    plsc.subcore_barrier()

    epc = e // NC
    epw = epc // NS
    base0 = c * epc + s * epw
    csz = src_sl.shape[0]

    nb = csz // kb
    zero16i = jnp.zeros((LANES,), jnp.int32)

    def process(rows_v, ssem, loff):
        for q in range(kb // LANES):
            goff = loff + q * LANES
            w16 = w_sl[pl.ds(goff, LANES)]
            for j in range(LANES):
                wj = jnp.full((LANES,), w16[j])
                r = q * LANES + j
                for ch in range(d // LANES):
                    sl = pl.ds(ch * LANES, LANES)
                    rows_v[r, sl] = rows_v[r, sl] * wj
            dst16 = dst_sl[pl.ds(goff, LANES)]
            cnt, _ = plsc.scan_count(dst16)
            has_dup = jnp.max(cnt) != jnp.min(cnt)

            def fast():
                dstw_v[q, :] = dst16
                pltpu.async_copy(rows_v.at[pl.ds(q * LANES, LANES)],
                                 acc_sh.at[dstw_v.at[q]], ssem, add=True)

            def slow():
                plsc.store_scatter(dstc_v.at[q], [iota, zero16i], dst16)
                for j in range(LANES):
                    pltpu.async_copy(rows_v.at[pl.ds(q * LANES + j, 1)],
                                     acc_sh.at[dstc_v.at[q, j]], ssem, add=True)

            lax.cond(has_dup, slow, fast)

    def prefetch(i, rows_v, sem):
        loff = jnp.minimum(i, nb - 1) * kb
        pltpu.async_copy(g_hbm.at[src_sl.at[pl.ds(loff, kb)]], rows_v, sem)

    def gwait(rows_v, sem):
        pltpu.make_async_copy(g_hbm.at[pl.ds(0, kb)], rows_v, sem).wait()

    def sdrain(rows_v, ssem):
        pltpu.make_async_copy(g_hbm.at[pl.ds(0, kb)], rows_v, ssem).wait()

    def cbody(ci, _):
        cb = base0 + ci * csz
        pltpu.sync_copy(src_hbm.at[pl.ds(cb, csz)], src_sl)
        pltpu.sync_copy(dst_hbm.at[pl.ds(cb, csz)], dst_sl)
        pltpu.sync_copy(w_hbm.at[pl.ds(cb, csz)], w_sl)
        prefetch(jnp.int32(0), rows0_v, sem0)

        def ebatch(i, _):
            @pl.when(i % 2 == 0)
            def _():
                @pl.when(i >= 1)
                def _():
                    sdrain(rows1_v, ssem1)

                prefetch(i + 1, rows1_v, sem1)
                gwait(rows0_v, sem0)
                process(rows0_v, ssem0, i * kb)

            @pl.when(i % 2 == 1)
            def _():
                sdrain(rows0_v, ssem0)
                prefetch(i + 1, rows0_v, sem0)
                gwait(rows1_v, sem1)
                process(rows1_v, ssem1, i * kb)

            return ()

        lax.fori_loop(0, nb, ebatch, ())
        if nb % 2 == 0:
            sdrain(rows1_v, ssem1)
            gwait(rows0_v, sem0)
        else:
            sdrain(rows0_v, ssem0)
            gwait(rows1_v, sem1)
        return ()

    lax.fori_loop(0, epw // csz, cbody, ())
    plsc.subcore_barrier()
    pltpu.sync_copy(
        acc_sh.at[pl.ds(s * rows_per_tile, rows_per_tile)],
        out_hbm.at[c, pl.ds(s * rows_per_tile, rows_per_tile)],
    )


def _edge_partials(src, dst, w, g):
    n, d = g.shape
    e = w.shape[0]
    kb = 80
    zrows = 32
    csz = 2000
    n_pad = ((n + 128 * NS - 1) // (128 * NS)) * (128 * NS)
    assert e % NC == 0 and (e // NC) % NS == 0 and (e // NW) % csz == 0
    assert csz % kb == 0 and (n_pad // NS) % zrows == 0
    mesh = plsc.VectorSubcoreMesh(core_axis_name="c", subcore_axis_name="s")
    k = pl.kernel(
        functools.partial(_edge_kernel_body, n_pad, d, e, kb),
        out_type=jax.ShapeDtypeStruct((NC, n_pad, d), jnp.float32),
        mesh=mesh,
        compiler_params=pltpu.CompilerParams(needs_layout_passes=False),
        scratch_types=[
            pltpu.VMEM((csz,), jnp.int32),
            pltpu.VMEM((csz,), jnp.int32),
            pltpu.VMEM((csz,), jnp.float32),
            pltpu.VMEM((kb, d), jnp.float32),
            pltpu.VMEM((kb, d), jnp.float32),
            pltpu.VMEM((kb // LANES, LANES), jnp.int32),
            pltpu.VMEM((kb // LANES, LANES, 1), jnp.int32),
            pltpu.VMEM((zrows, d), jnp.float32),
            pltpu.VMEM_SHARED((n_pad, d), jnp.float32),
            pltpu.SemaphoreType.DMA,
            pltpu.SemaphoreType.DMA,
            pltpu.SemaphoreType.DMA,
            pltpu.SemaphoreType.DMA,
        ],
    )
    return k(src, dst, w, g)


def _combine_body(n, p_ref, g_ref, deg_ref, b_ref, o_ref):
    deg = jnp.sum(deg_ref[...], axis=0) + 1.0
    dinv = lax.rsqrt(deg)
    ssum = p_ref[0, pl.ds(0, n), :] + p_ref[1, pl.ds(0, n), :] + g_ref[...]
    o_ref[...] = ssum * dinv[:, None] + b_ref[...]


def _combine(p, g, deg_parts, b):
    n, d = g.shape
    return pl.pallas_call(
        functools.partial(_combine_body, n),
        out_shape=jax.ShapeDtypeStruct((n, d), jnp.float32),
    )(p, g, deg_parts, b)


def kernel(x, edge_index, weights, emb_table, W, b):
    n, d = emb_table.shape
    src = edge_index[0]
    dst = edge_index[1]
    deg_parts = _deg_partials(dst, weights, n)
    g = _matmul_g(emb_table, W, deg_parts)
    p = _edge_partials(src, dst, weights, g)
    out = _combine(p, g, deg_parts, b.reshape(1, d))
    return out[None, :, None, :]
